# baseline (speedup 1.0000x reference)
import functools

import jax
import jax.numpy as jnp
from jax import lax
from jax.experimental import pallas as pl
from jax.experimental.pallas import tpu as pltpu
from jax.experimental.pallas import tpu_sc as plsc

B = 16
L = 4096
D = 128
NC = 2
NS = 16
HALF = L // 2
REP = 256
NIDX = 3 * REP // 128


def _body(x_hbm, table_hbm, idx_hbm, out_hbm, xrow_v, const_v, idx_v, sem):
    wid = lax.axis_index("s") * NC + lax.axis_index("c")
    b = wid >> 1
    h = wid & 1

    pltpu.sync_copy(idx_hbm, idx_v)
    gathers = []
    for j in range(NIDX):
        gathers.append(
            pltpu.async_copy(
                table_hbm.at[idx_v.at[j]],
                const_v.at[pl.ds(j * 128, 128)],
                sem,
            )
        )

    pltpu.sync_copy(x_hbm.at[b], xrow_v)
    lanes = lax.iota(jnp.int32, 16)
    big = jnp.full((16,), L, jnp.int32)

    def scan_step(i, carry):
        a4, a2 = carry
        v = xrow_v[pl.ds(i * 16, 16)]
        pos = lanes + i * 16
        a4 = jnp.minimum(a4, jnp.where(v == 4, pos, L))
        a2 = jnp.minimum(a2, jnp.where(v == 2, pos, L))
        return a4, a2

    a4, a2 = lax.fori_loop(0, L // 16, scan_step, (big, big))
    sep = jnp.min(a4)
    eos = jnp.min(a2)

    t0 = h * HALF
    t1 = t0 + HALF
    s = jnp.minimum(jnp.maximum(sep, t0), t1)
    e = jnp.minimum(jnp.maximum(eos, s), t1)

    for g in gathers:
        g.wait()

    def copy_run(rid, a0, a1):
        ln = a1 - a0

        def chunk(i, _):
            pltpu.sync_copy(
                const_v.at[pl.ds(rid * REP, REP)],
                out_hbm.at[b, pl.ds(a0 + (i << 8), REP)],
            )
            return 0

        lax.fori_loop(0, ln >> 8, chunk, 0)
        rem = ln & (REP - 1)
        base = a0 + (ln - rem)
        for bp in range(7, -1, -1):
            k = 1 << bp
            off = base + ((rem >> (bp + 1)) << (bp + 1))

            @pl.when(((rem >> bp) & 1) == 1)
            def _(k=k, off=off):
                pltpu.sync_copy(
                    const_v.at[pl.ds(rid * REP, k)],
                    out_hbm.at[b, pl.ds(off, k)],
                )

    copy_run(1, t0, s)
    copy_run(2, s, e)
    copy_run(0, e, t1)


_sc_call = pl.kernel(
    _body,
    out_type=jax.ShapeDtypeStruct((B, L, D), jnp.float32),
    mesh=plsc.VectorSubcoreMesh(core_axis_name="c", subcore_axis_name="s"),
    compiler_params=pltpu.CompilerParams(
        use_tc_tiling_on_sc=False, needs_layout_passes=False
    ),
    scratch_types=[
        pltpu.VMEM((L,), jnp.int32),
        pltpu.VMEM((3 * REP, D), jnp.float32),
        pltpu.VMEM((NIDX, 128), jnp.int32),
        pltpu.SemaphoreType.DMA,
    ],
)


@jax.jit
def kernel(x, seg_table):
    idx = jnp.repeat(jnp.arange(3, dtype=jnp.int32), REP).reshape(NIDX, 128)
    return _sc_call(x, seg_table, idx)


# speedup vs baseline: 21.1944x; 21.1944x over previous
import jax
import jax.numpy as jnp
from jax import lax
from jax.experimental import pallas as pl
from jax.experimental.pallas import tpu as pltpu
from jax.experimental.pallas import tpu_sc as plsc

B = 16
L = 4096
D = 128
NC = 2
NS = 16
HALF = L // 2
REP = 128
REP_LOG2 = 7


def _body(x_hbm, table_hbm, out_hbm, xrow_v, table_v, const_v, sem):
    wid = lax.axis_index("s") * NC + lax.axis_index("c")
    b = wid >> 1
    h = wid & 1

    pltpu.sync_copy(table_hbm, table_v)
    pltpu.sync_copy(x_hbm.at[b], xrow_v)

    rowvecs = [
        [table_v[r, pl.ds(j * 16, 16)] for j in range(D // 16)]
        for r in range(3)
    ]

    def fill_step(i, _):
        for r in range(3):
            for j in range(D // 16):
                const_v[r * REP + i, pl.ds(j * 16, 16)] = rowvecs[r][j]
        return 0

    lax.fori_loop(0, REP, fill_step, 0)

    lanes = lax.iota(jnp.int32, 16)
    big = jnp.full((16,), L, jnp.int32)

    def scan_step(i, carry):
        a4, a2 = carry
        v = xrow_v[pl.ds(i * 16, 16)]
        pos = lanes + i * 16
        a4 = jnp.minimum(a4, jnp.where(v == 4, pos, L))
        a2 = jnp.minimum(a2, jnp.where(v == 2, pos, L))
        return a4, a2

    a4, a2 = lax.fori_loop(0, L // 16, scan_step, (big, big))
    sep = jnp.min(a4)
    eos = jnp.min(a2)

    t0 = h * HALF
    t1 = t0 + HALF
    s = jnp.minimum(jnp.maximum(sep, t0), t1)
    e = jnp.minimum(jnp.maximum(eos, s), t1)

    def copy_run(rid, a0, a1, fire):
        ln = a1 - a0

        def chunk(i, _):
            src = const_v.at[pl.ds(rid * REP, REP)]
            dst = out_hbm.at[b, pl.ds(a0 + (i << REP_LOG2), REP)]
            if fire:
                pltpu.async_copy(src, dst, sem)
            else:
                pltpu.make_async_copy(src, dst, sem).wait()
            return 0

        lax.fori_loop(0, ln >> REP_LOG2, chunk, 0)
        rem = ln & (REP - 1)
        base = a0 + (ln - rem)
        for bp in range(REP_LOG2 - 1, -1, -1):
            k = 1 << bp
            off = base + ((rem >> (bp + 1)) << (bp + 1))

            @pl.when(((rem >> bp) & 1) == 1)
            def _(k=k, off=off):
                src = const_v.at[pl.ds(rid * REP, k)]
                dst = out_hbm.at[b, pl.ds(off, k)]
                if fire:
                    pltpu.async_copy(src, dst, sem)
                else:
                    pltpu.make_async_copy(src, dst, sem).wait()

    for fire in (True, False):
        copy_run(1, t0, s, fire)
        copy_run(2, s, e, fire)
        copy_run(0, e, t1, fire)


_sc_call = pl.kernel(
    _body,
    out_type=jax.ShapeDtypeStruct((B, L, D), jnp.float32),
    mesh=plsc.VectorSubcoreMesh(core_axis_name="c", subcore_axis_name="s"),
    compiler_params=pltpu.CompilerParams(
        use_tc_tiling_on_sc=False, needs_layout_passes=False
    ),
    scratch_types=[
        pltpu.VMEM((L,), jnp.int32),
        pltpu.VMEM((3, D), jnp.float32),
        pltpu.VMEM((3 * REP, D), jnp.float32),
        pltpu.SemaphoreType.DMA,
    ],
)


@jax.jit
def kernel(x, seg_table):
    return _sc_call(x, seg_table)


# speedup vs baseline: 21.9038x; 1.0335x over previous
import jax
import jax.numpy as jnp
from jax import lax
from jax.experimental import pallas as pl
from jax.experimental.pallas import tpu as pltpu
from jax.experimental.pallas import tpu_sc as plsc

B = 16
L = 4096
D = 128
NC = 2
NS = 16
HALF = L // 2
REP = 128
REP_LOG2 = 7


def _body(x_hbm, table_hbm, out_hbm, xrow_v, table_v, const_v, sem):
    wid = lax.axis_index("s") * NC + lax.axis_index("c")
    b = wid >> 1
    h = wid & 1

    pltpu.sync_copy(table_hbm, table_v)
    pltpu.sync_copy(x_hbm.at[b], xrow_v)

    rowvecs = [
        [table_v[r, pl.ds(j * 16, 16)] for j in range(D // 16)]
        for r in range(3)
    ]
    lanes = lax.iota(jnp.int32, 16)
    big = jnp.full((16,), L, jnp.int32)

    def fused_step(i, carry):
        a4, a2 = carry
        for r in range(3):
            for j in range(D // 16):
                const_v[r * REP + i, pl.ds(j * 16, 16)] = rowvecs[r][j]
        for u in range(L // 16 // REP):
            c = i * (L // 16 // REP) + u
            v = xrow_v[pl.ds(c * 16, 16)]
            pos = lanes + c * 16
            a4 = jnp.minimum(a4, jnp.where(v == 4, pos, L))
            a2 = jnp.minimum(a2, jnp.where(v == 2, pos, L))
        return a4, a2

    a4, a2 = lax.fori_loop(0, REP, fused_step, (big, big))
    sep = jnp.min(a4)
    eos = jnp.min(a2)

    t0 = h * HALF
    t1 = t0 + HALF
    s = jnp.minimum(jnp.maximum(sep, t0), t1)
    e = jnp.minimum(jnp.maximum(eos, s), t1)

    def copy_run(rid, a0, a1):
        ln = a1 - a0

        def chunk(i, _):
            pltpu.async_copy(
                const_v.at[pl.ds(rid * REP, REP)],
                out_hbm.at[b, pl.ds(a0 + (i << REP_LOG2), REP)],
                sem,
            )
            return 0

        lax.fori_loop(0, ln >> REP_LOG2, chunk, 0)
        rem = ln & (REP - 1)
        base = a0 + (ln - rem)
        for bp in range(REP_LOG2 - 1, -1, -1):
            k = 1 << bp
            off = base + ((rem >> (bp + 1)) << (bp + 1))

            @pl.when(((rem >> bp) & 1) == 1)
            def _(k=k, off=off):
                pltpu.async_copy(
                    const_v.at[pl.ds(rid * REP, k)],
                    out_hbm.at[b, pl.ds(off, k)],
                    sem,
                )

    copy_run(1, t0, s)
    copy_run(2, s, e)
    copy_run(0, e, t1)

    span = out_hbm.at[b, pl.ds(t0, HALF)]
    pltpu.make_async_copy(span, span, sem).wait()


_sc_call = pl.kernel(
    _body,
    out_type=jax.ShapeDtypeStruct((B, L, D), jnp.float32),
    mesh=plsc.VectorSubcoreMesh(core_axis_name="c", subcore_axis_name="s"),
    compiler_params=pltpu.CompilerParams(
        use_tc_tiling_on_sc=False, needs_layout_passes=False
    ),
    scratch_types=[
        pltpu.VMEM((L,), jnp.int32),
        pltpu.VMEM((3, D), jnp.float32),
        pltpu.VMEM((3 * REP, D), jnp.float32),
        pltpu.SemaphoreType.DMA,
    ],
)


@jax.jit
def kernel(x, seg_table):
    return _sc_call(x, seg_table)


# speedup vs baseline: 23.1531x; 1.0570x over previous
import jax
import jax.numpy as jnp
from jax import lax
from jax.experimental import pallas as pl
from jax.experimental.pallas import tpu as pltpu
from jax.experimental.pallas import tpu_sc as plsc

B = 16
L = 4096
D = 128
NC = 2
NS = 16
HALF = L // 2
REP = 128
REP_LOG2 = 7


def _body(x_hbm, table_hbm, out_hbm, xrow_v, table_v, const_v, sem):
    wid = lax.axis_index("s") * NC + lax.axis_index("c")
    b = wid >> 1
    h = wid & 1

    cp_t = pltpu.async_copy(table_hbm, table_v, sem)
    cp_x = pltpu.async_copy(x_hbm.at[b], xrow_v, sem)
    cp_t.wait()
    cp_x.wait()

    rowvecs = [
        [table_v[r, pl.ds(j * 16, 16)] for j in range(D // 16)]
        for r in range(3)
    ]
    lanes = lax.iota(jnp.int32, 16)
    big = jnp.full((16,), L, jnp.int32)

    def fused_step(i, carry):
        a4, a2 = carry
        for r in range(3):
            for j in range(D // 16):
                const_v[r * REP + i, pl.ds(j * 16, 16)] = rowvecs[r][j]
        for u in range(L // 16 // REP):
            c = i * (L // 16 // REP) + u
            v = xrow_v[pl.ds(c * 16, 16)]
            pos = lanes + c * 16
            a4 = jnp.minimum(a4, jnp.where(v == 4, pos, L))
            a2 = jnp.minimum(a2, jnp.where(v == 2, pos, L))
        return a4, a2

    a4, a2 = lax.fori_loop(0, REP, fused_step, (big, big))
    sep = jnp.min(a4)
    eos = jnp.min(a2)

    t0 = h * HALF
    t1 = t0 + HALF
    s = jnp.minimum(jnp.maximum(sep, t0), t1)
    e = jnp.minimum(jnp.maximum(eos, s), t1)

    def copy_run(rid, a0, a1):
        ln = a1 - a0

        def chunk(i, _):
            pltpu.async_copy(
                const_v.at[pl.ds(rid * REP, REP)],
                out_hbm.at[b, pl.ds(a0 + (i << REP_LOG2), REP)],
                sem,
            )
            return 0

        lax.fori_loop(0, ln >> REP_LOG2, chunk, 0)
        rem = ln & (REP - 1)
        base = a0 + (ln - rem)
        for bp in range(REP_LOG2 - 1, -1, -1):
            k = 1 << bp
            off = base + ((rem >> (bp + 1)) << (bp + 1))

            @pl.when(((rem >> bp) & 1) == 1)
            def _(k=k, off=off):
                pltpu.async_copy(
                    const_v.at[pl.ds(rid * REP, k)],
                    out_hbm.at[b, pl.ds(off, k)],
                    sem,
                )

    copy_run(1, t0, s)
    copy_run(2, s, e)
    copy_run(0, e, t1)

    span = out_hbm.at[b, pl.ds(t0, HALF)]
    pltpu.make_async_copy(span, span, sem).wait()


_sc_call = pl.kernel(
    _body,
    out_type=jax.ShapeDtypeStruct((B, L, D), jnp.float32),
    mesh=plsc.VectorSubcoreMesh(core_axis_name="c", subcore_axis_name="s"),
    compiler_params=pltpu.CompilerParams(
        use_tc_tiling_on_sc=False, needs_layout_passes=False
    ),
    scratch_types=[
        pltpu.VMEM((L,), jnp.int32),
        pltpu.VMEM((3, D), jnp.float32),
        pltpu.VMEM((3 * REP, D), jnp.float32),
        pltpu.SemaphoreType.DMA,
    ],
)


@jax.jit
def kernel(x, seg_table):
    return _sc_call(x, seg_table)


# speedup vs baseline: 23.9251x; 1.0333x over previous
import jax
import jax.numpy as jnp
from jax import lax
from jax.experimental import pallas as pl
from jax.experimental.pallas import tpu as pltpu
from jax.experimental.pallas import tpu_sc as plsc

B = 16
L = 4096
D = 128
NC = 2
NS = 16
HALF = L // 2
REP = 64
REP_LOG2 = 6


def _body(x_hbm, table_hbm, out_hbm, xrow_v, table_v, const_v, sem):
    wid = lax.axis_index("s") * NC + lax.axis_index("c")
    b = wid >> 1
    h = wid & 1

    cp_t = pltpu.async_copy(table_hbm, table_v, sem)
    cp_x = pltpu.async_copy(x_hbm.at[b], xrow_v, sem)
    cp_t.wait()
    cp_x.wait()

    rowvecs = [
        [table_v[r, pl.ds(j * 16, 16)] for j in range(D // 16)]
        for r in range(3)
    ]
    lanes = lax.iota(jnp.int32, 16)
    big = jnp.full((16,), L, jnp.int32)

    def fused_step(i, carry):
        a4, a2 = carry
        for r in range(3):
            for j in range(D // 16):
                const_v[r * REP + i, pl.ds(j * 16, 16)] = rowvecs[r][j]
        for u in range(L // 16 // REP):
            c = i * (L // 16 // REP) + u
            v = xrow_v[pl.ds(c * 16, 16)]
            pos = lanes + c * 16
            a4 = jnp.minimum(a4, jnp.where(v == 4, pos, L))
            a2 = jnp.minimum(a2, jnp.where(v == 2, pos, L))
        return a4, a2

    a4, a2 = lax.fori_loop(0, REP, fused_step, (big, big))
    sep = jnp.min(a4)
    eos = jnp.min(a2)

    t0 = h * HALF
    t1 = t0 + HALF
    s = jnp.minimum(jnp.maximum(sep, t0), t1)
    e = jnp.minimum(jnp.maximum(eos, s), t1)

    def copy_run(rid, a0, a1):
        ln = a1 - a0

        def chunk(i, _):
            pltpu.async_copy(
                const_v.at[pl.ds(rid * REP, REP)],
                out_hbm.at[b, pl.ds(a0 + (i << REP_LOG2), REP)],
                sem,
            )
            return 0

        lax.fori_loop(0, ln >> REP_LOG2, chunk, 0)
        rem = ln & (REP - 1)
        base = a0 + (ln - rem)
        for bp in range(REP_LOG2 - 1, -1, -1):
            k = 1 << bp
            off = base + ((rem >> (bp + 1)) << (bp + 1))

            @pl.when(((rem >> bp) & 1) == 1)
            def _(k=k, off=off):
                pltpu.async_copy(
                    const_v.at[pl.ds(rid * REP, k)],
                    out_hbm.at[b, pl.ds(off, k)],
                    sem,
                )

    copy_run(1, t0, s)
    copy_run(2, s, e)
    copy_run(0, e, t1)

    span = out_hbm.at[b, pl.ds(t0, HALF)]
    pltpu.make_async_copy(span, span, sem).wait()


_sc_call = pl.kernel(
    _body,
    out_type=jax.ShapeDtypeStruct((B, L, D), jnp.float32),
    mesh=plsc.VectorSubcoreMesh(core_axis_name="c", subcore_axis_name="s"),
    compiler_params=pltpu.CompilerParams(
        use_tc_tiling_on_sc=False, needs_layout_passes=False
    ),
    scratch_types=[
        pltpu.VMEM((L,), jnp.int32),
        pltpu.VMEM((3, D), jnp.float32),
        pltpu.VMEM((3 * REP, D), jnp.float32),
        pltpu.SemaphoreType.DMA,
    ],
)


@jax.jit
def kernel(x, seg_table):
    return _sc_call(x, seg_table)
